# baseline (speedup 1.0000x reference)
import jax
import jax.numpy as jnp
from jax import lax
from jax.experimental import pallas as pl
from jax.experimental.pallas import tpu as pltpu
from jax.experimental.pallas import tpu_sc as plsc

_B, _N, _M, _K, _C = 4, 8192, 1024, 32, 128
_R2 = 0.2 * 0.2
_NC, _NS, _L = 2, 16, 16
_TPB = (_NC * _NS) // _B
_MPT = _M // _TPB
_MK = _M * _K
_NCH = 3 + _C
_PLANE = _K * _M

_PX, _PY, _PZ = 0, _N, 2 * _N
_CX = 3 * _N

_mesh = plsc.VectorSubcoreMesh(core_axis_name="c", subcore_axis_name="s")


def _body(ptsT, cenT, ff, idx_out, out, frows, idxv, obuf, pbuf, pbuf2, sem_out):
    wid = lax.axis_index("c") * _NS + lax.axis_index("s")
    b = wid // _TPB
    cg = wid % _TPB
    ms = cg * _MPT

    iota = lax.iota(jnp.int32, _L)
    zeros = jnp.zeros((_L,), jnp.int32)
    full = jnp.ones((_L,), jnp.bool_)

    pltpu.sync_copy(ptsT.at[pl.ds((b * 3 + 0) * _N, _N)], frows.at[pl.ds(_PX, _N)])
    pltpu.sync_copy(ptsT.at[pl.ds((b * 3 + 1) * _N, _N)], frows.at[pl.ds(_PY, _N)])
    pltpu.sync_copy(ptsT.at[pl.ds((b * 3 + 2) * _N, _N)], frows.at[pl.ds(_PZ, _N)])
    for c3 in range(3):
        pltpu.sync_copy(cenT.at[pl.ds((b * 3 + c3) * _M + ms, _MPT)],
                        frows.at[pl.ds(_CX + c3 * _MPT, _MPT)])

    physk = [((iota + h * _L) >> 3) * 1024 + ((iota + h * _L) & 7) * 128
             for h in range(_K // _L)]
    kmul = [(iota + h * _L) * _MPT for h in range(_K // _L)]

    def scan_pts(pos, cxv, cyv, czv):
        dx = frows[pl.ds(_PX + pos, _L)] - cxv
        dy = frows[pl.ds(_PY + pos, _L)] - cyv
        dz = frows[pl.ds(_PZ + pos, _L)] - czv
        return dx * dx + dy * dy + dz * dz < _R2

    def fixup(ml, cnt, base, cxv, cyv, czv):
        first = plsc.load_gather(pbuf, [jnp.full((_L,), base, jnp.int32)])
        for h in range(_K // _L):
            kv = iota + h * _L
            v = pbuf[pl.ds(base + h * _L, _L)]
            v = jnp.where(kv < cnt, v, first)
            gx = plsc.load_gather(frows, [v]) - cxv
            gy = plsc.load_gather(frows, [v + _N]) - cyv
            gz = plsc.load_gather(frows, [v + 2 * _N]) - czv
            addr = physk[h] + ml
            plsc.store_scatter(obuf, [addr], gx)
            plsc.store_scatter(obuf, [addr + 4096], gy)
            plsc.store_scatter(obuf, [addr + 8192], gz)
            tv = ((v >> 7) << 10) + (v & 127)
            plsc.store_scatter(pbuf2, [kmul[h] + ml], tv)

    def per_quad(pi, carry):
        mls = [4 * pi + j for j in range(4)]
        cs = []
        for ml in mls:
            cs.append((
                plsc.load_gather(frows, [jnp.full((_L,), _CX + ml, jnp.int32)]),
                plsc.load_gather(frows, [jnp.full((_L,), _CX + _MPT + ml,
                                                  jnp.int32)]),
                plsc.load_gather(frows, [jnp.full((_L,), _CX + 2 * _MPT + ml,
                                                  jnp.int32)]),
            ))
        for j in range(4):
            pbuf[pl.ds(64 * j, _L)] = zeros

        def cond(st):
            pos = st[0]
            unfinished = (st[1] < _K) | (st[2] < _K) | (st[3] < _K) | (st[4] < _K)
            return jnp.logical_and(unfinished, pos < _N)

        def body(st):
            pos = st[0]
            cnts = list(st[1:])
            for h in range(2):
                ih = iota + (pos + h * _L)
                ms = [scan_pts(pos + h * _L, *cs[j]) for j in range(4)]
                for j in range(4):
                    plsc.store_compressed(
                        pbuf.at[pl.ds(64 * j + jnp.minimum(cnts[j], 48), _L)],
                        ih, mask=ms[j])
                for j in range(4):
                    cnts[j] = cnts[j] + plsc.all_reduce_population_count(ms[j])[0]
            return (pos + 2 * _L, *cnts)

        st = lax.while_loop(
            cond, body, (jnp.int32(0),) + (jnp.int32(0),) * 4)

        for j in range(4):
            fixup(mls[j], st[1 + j], 64 * j, *cs[j])
        return carry

    lax.fori_loop(0, _MPT // 4, per_quad, 0)

    pltpu.sync_copy(pbuf2, idx_out.at[pl.ds((b * _TPB + cg) * 4096, 4096)])
    for c3 in range(3):
        for tk in range(4):
            dst = (b * _NCH + c3) * _PLANE + tk * 8192 + cg * 1024
            pltpu.sync_copy(obuf.at[pl.ds(c3 * 4096 + tk * 1024, 1024)],
                            out.at[pl.ds(dst, 1024)])

    plsc.subcore_barrier()

    c0 = cg * 16

    def tp_body(tp, carry):
        t = tp >> 1
        p = tp & 1

        @pl.when(p == 0)
        def _():
            pltpu.sync_copy(
                ff.at[pl.ds((b * 16 + cg * 2 + t) * 8 * _N, 8 * _N)], frows)

        us = [(p * 4 + q) * 128 for q in range(4)]

        def ci2_body(ci2, carry2):
            @pl.when((ci2 & 1) == 0)
            def _():
                pltpu.sync_copy(
                    idx_out.at[pl.ds(b * 8 * 4096 + (ci2 >> 1) * 16384,
                                     16384)], idxv)

            @pl.when(jnp.logical_or(tp > 0, ci2 > 0))
            def _():
                for _q in range(32):
                    pltpu.make_async_copy(
                        obuf.at[pl.ds(0, 1024)],
                        out.at[pl.ds(0, 1024)],
                        sem_out).wait()

            for half in range(2):
                ci = 2 * ci2 + half
                ib0 = (ci2 & 1) * 8192 + half * 4096
                ob0 = half * 16384

                iv = idxv[pl.ds(ib0, _L)]
                for k in range(_K):
                    kb = ib0 + k * _MPT
                    sb = ob0 + (k >> 3) * 1024 + (k & 7) * 128
                    for mseg in range(8):
                        vals = [plsc.load_gather(frows, [iv + us[q]])
                                for q in range(4)]
                        if mseg < 7:
                            iv = idxv[pl.ds(kb + (mseg + 1) * _L, _L)]
                        elif k < _K - 1:
                            iv = idxv[pl.ds(kb + _MPT, _L)]
                        for q in range(4):
                            obuf[pl.ds(sb + q * 4096 + mseg * _L, _L)] = vals[q]

                for q in range(4):
                    ch = c0 + t * 8 + p * 4 + q
                    pb = (b * _NCH + 3 + ch) * _PLANE + ci * 1024
                    for tk in range(4):
                        pltpu.async_copy(
                            obuf.at[pl.ds(ob0 + q * 4096 + tk * 1024, 1024)],
                            out.at[pl.ds(pb + tk * 8192, 1024)],
                            sem_out)
            return carry2

        lax.fori_loop(0, 4, ci2_body, 0)
        return carry

    lax.fori_loop(0, 4, tp_body, 0)
    for _q in range(32):
        pltpu.make_async_copy(
            obuf.at[pl.ds(0, 1024)],
            out.at[pl.ds(0, 1024)],
            sem_out).wait()


_fused = pl.kernel(
    _body,
    out_type=(
        jax.ShapeDtypeStruct((_B * _TPB * 4096,), jnp.int32),
        jax.ShapeDtypeStruct((_B * _NCH * _PLANE,), jnp.float32),
    ),
    mesh=_mesh,
    compiler_params=pltpu.CompilerParams(needs_layout_passes=False,
                                         disable_bounds_checks=True),
    scratch_types=[
        pltpu.VMEM((8 * _N,), jnp.float32),
        pltpu.VMEM((16384,), jnp.int32),
        pltpu.VMEM((32768,), jnp.float32),
        pltpu.VMEM((256,), jnp.int32),
        pltpu.VMEM((4096,), jnp.int32),
        pltpu.SemaphoreType.DMA,
    ],
)


def kernel(points_xyz, center_xyz, features):
    ptsT = jnp.transpose(points_xyz, (0, 2, 1)).reshape(-1)
    cenT = jnp.transpose(center_xyz, (0, 2, 1)).reshape(-1)
    ff = (features.reshape(_B, 16, 8, 64, 128)
          .transpose(0, 1, 3, 2, 4).reshape(-1))
    _, out = _fused(ptsT, cenT, ff)
    o6 = out.reshape(_B, _NCH, 4, 8, 8, 128)
    return o6.transpose(0, 1, 3, 5, 2, 4).reshape(_B, _NCH, _M, _K)


# speedup vs baseline: 1.3613x; 1.3613x over previous
import jax
import jax.numpy as jnp
from jax import lax
from jax.experimental import pallas as pl
from jax.experimental.pallas import tpu as pltpu
from jax.experimental.pallas import tpu_sc as plsc

_B, _N, _M, _K, _C = 4, 8192, 1024, 32, 128
_R2 = 0.2 * 0.2
_NC, _NS, _L = 2, 16, 16
_TPB = (_NC * _NS) // _B
_MPT = _M // _TPB
_MK = _M * _K
_NCH = 3 + _C
_PLANE = _K * _M

_PX, _PY, _PZ = 0, _N, 2 * _N
_CX = 3 * _N

_mesh = plsc.VectorSubcoreMesh(core_axis_name="c", subcore_axis_name="s")


def _body(ptsT, cenT, ff, idx_out, out, frows, idxv, obuf, pbuf, pbuf2, sem_out):
    wid = lax.axis_index("c") * _NS + lax.axis_index("s")
    b = wid // _TPB
    cg = wid % _TPB
    ms = cg * _MPT

    iota = lax.iota(jnp.int32, _L)
    zeros = jnp.zeros((_L,), jnp.int32)
    full = jnp.ones((_L,), jnp.bool_)

    pltpu.sync_copy(ptsT.at[pl.ds((b * 3 + 0) * _N, _N)], frows.at[pl.ds(_PX, _N)])
    pltpu.sync_copy(ptsT.at[pl.ds((b * 3 + 1) * _N, _N)], frows.at[pl.ds(_PY, _N)])
    pltpu.sync_copy(ptsT.at[pl.ds((b * 3 + 2) * _N, _N)], frows.at[pl.ds(_PZ, _N)])
    for c3 in range(3):
        pltpu.sync_copy(cenT.at[pl.ds((b * 3 + c3) * _M + ms, _MPT)],
                        frows.at[pl.ds(_CX + c3 * _MPT, _MPT)])

    physk = [((iota + h * _L) >> 3) * 1024 + ((iota + h * _L) & 7) * 128
             for h in range(_K // _L)]
    kmul = [(iota + h * _L) * _MPT for h in range(_K // _L)]

    def scan_pts(pos, cxv, cyv, czv):
        dx = frows[pl.ds(_PX + pos, _L)] - cxv
        dy = frows[pl.ds(_PY + pos, _L)] - cyv
        dz = frows[pl.ds(_PZ + pos, _L)] - czv
        return dx * dx + dy * dy + dz * dz < _R2

    def fixup(ml, cnt, base, cxv, cyv, czv):
        first = plsc.load_gather(pbuf, [jnp.full((_L,), base, jnp.int32)])
        for h in range(_K // _L):
            kv = iota + h * _L
            v = pbuf[pl.ds(base + h * _L, _L)]
            v = jnp.where(kv < cnt, v, first)
            gx = plsc.load_gather(frows, [v]) - cxv
            gy = plsc.load_gather(frows, [v + _N]) - cyv
            gz = plsc.load_gather(frows, [v + 2 * _N]) - czv
            addr = physk[h] + ml
            plsc.store_scatter(obuf, [addr], gx)
            plsc.store_scatter(obuf, [addr + 4096], gy)
            plsc.store_scatter(obuf, [addr + 8192], gz)
            tv = ((v >> 7) << 10) + (v & 127)
            plsc.store_scatter(pbuf2, [kmul[h] + ml], tv)

    def per_quad(pi, carry):
        mls = [4 * pi + j for j in range(4)]
        cs = []
        for ml in mls:
            cs.append((
                plsc.load_gather(frows, [jnp.full((_L,), _CX + ml, jnp.int32)]),
                plsc.load_gather(frows, [jnp.full((_L,), _CX + _MPT + ml,
                                                  jnp.int32)]),
                plsc.load_gather(frows, [jnp.full((_L,), _CX + 2 * _MPT + ml,
                                                  jnp.int32)]),
            ))
        for j in range(4):
            pbuf[pl.ds(64 * j, _L)] = zeros

        def cond(st):
            pos = st[0]
            unfinished = (st[1] < _K) | (st[2] < _K) | (st[3] < _K) | (st[4] < _K)
            return jnp.logical_and(unfinished, pos < _N)

        def body(st):
            pos = st[0]
            cnts = list(st[1:])
            for h in range(2):
                ih = iota + (pos + h * _L)
                ms = [scan_pts(pos + h * _L, *cs[j]) for j in range(4)]
                for j in range(4):
                    plsc.store_compressed(
                        pbuf.at[pl.ds(64 * j + jnp.minimum(cnts[j], 48), _L)],
                        ih, mask=ms[j])
                for j in range(4):
                    cnts[j] = cnts[j] + plsc.all_reduce_population_count(ms[j])[0]
            return (pos + 2 * _L, *cnts)

        st = lax.while_loop(
            cond, body, (jnp.int32(0),) + (jnp.int32(0),) * 4)

        for j in range(4):
            fixup(mls[j], st[1 + j], 64 * j, *cs[j])
        return carry

    lax.fori_loop(0, _MPT // 4, per_quad, 0)

    pltpu.sync_copy(pbuf2, idx_out.at[pl.ds((b * _TPB + cg) * 4096, 4096)])
    for c3 in range(3):
        for tk in range(4):
            dst = (b * _NCH + c3) * _PLANE + tk * 8192 + cg * 1024
            pltpu.sync_copy(obuf.at[pl.ds(c3 * 4096 + tk * 1024, 1024)],
                            out.at[pl.ds(dst, 1024)])

    plsc.subcore_barrier()

    c0 = cg * 16

    for t in range(2):
        band = cg * 2 + t
        pltpu.sync_copy(ff.at[pl.ds((b * 16 + band) * 8 * _N, 8 * _N)], frows)
        for p in range(2):
            fr = [frows.at[pl.ds((p * 4 + q) * 128,
                                 8 * _N - (p * 4 + q) * 128)]
                  for q in range(4)]

            def chunk(ci, carry):
                par = ci & 1

                @pl.when((ci & 3) == 0)
                def _():
                    pltpu.sync_copy(
                        idx_out.at[pl.ds(b * 8 * 4096 + (ci >> 2) * 16384,
                                         16384)], idxv)

                @pl.when(ci >= 2)
                def _():
                    for _q in range(16):
                        pltpu.make_async_copy(
                            obuf.at[pl.ds(0, 1024)],
                            out.at[pl.ds(0, 1024)],
                            sem_out).wait()

                ib0 = (ci & 3) * 4096
                ob0 = par * 16384

                def kbody(k, carry2):
                    kb = ib0 + k * _MPT
                    sb = ob0 + (k >> 3) * 1024 + (k & 7) * 128
                    iv = plsc.load_expanded(idxv.at[pl.ds(kb, _L)], mask=full)
                    for mseg in range(8):
                        vals = [plsc.load_gather(fr[q], [iv]) for q in range(4)]
                        if mseg < 7:
                            iv = plsc.load_expanded(
                                idxv.at[pl.ds(kb + (mseg + 1) * _L, _L)],
                                mask=full)
                        for q in range(4):
                            plsc.store_compressed(
                                obuf.at[pl.ds(sb + q * 4096 + mseg * _L, _L)],
                                vals[q], mask=full)
                    return carry2

                lax.fori_loop(0, _K, kbody, 0)

                for q in range(4):
                    ch = c0 + t * 8 + p * 4 + q
                    pb = (b * _NCH + 3 + ch) * _PLANE + ci * 1024
                    for tk in range(4):
                        pltpu.async_copy(
                            obuf.at[pl.ds(ob0 + q * 4096 + tk * 1024, 1024)],
                            out.at[pl.ds(pb + tk * 8192, 1024)],
                            sem_out)
                return carry

            lax.fori_loop(0, _M // _MPT, chunk, 0)
            for _q in range(32):
                pltpu.make_async_copy(
                    obuf.at[pl.ds(0, 1024)],
                    out.at[pl.ds(0, 1024)],
                    sem_out).wait()


_fused = pl.kernel(
    _body,
    out_type=(
        jax.ShapeDtypeStruct((_B * _TPB * 4096,), jnp.int32),
        jax.ShapeDtypeStruct((_B * _NCH * _PLANE,), jnp.float32),
    ),
    mesh=_mesh,
    compiler_params=pltpu.CompilerParams(needs_layout_passes=False,
                                         disable_bounds_checks=True),
    scratch_types=[
        pltpu.VMEM((8 * _N,), jnp.float32),
        pltpu.VMEM((16384,), jnp.int32),
        pltpu.VMEM((32768,), jnp.float32),
        pltpu.VMEM((256,), jnp.int32),
        pltpu.VMEM((4096,), jnp.int32),
        pltpu.SemaphoreType.DMA,
    ],
)


def kernel(points_xyz, center_xyz, features):
    ptsT = jnp.transpose(points_xyz, (0, 2, 1)).reshape(-1)
    cenT = jnp.transpose(center_xyz, (0, 2, 1)).reshape(-1)
    ff = (features.reshape(_B, 16, 8, 64, 128)
          .transpose(0, 1, 3, 2, 4).reshape(-1))
    _, out = _fused(ptsT, cenT, ff)
    o6 = out.reshape(_B, _NCH, 4, 8, 8, 128)
    return o6.transpose(0, 1, 3, 5, 2, 4).reshape(_B, _NCH, _M, _K)
